# Initial kernel scaffold; baseline (speedup 1.0000x reference)
import jax
import jax.numpy as jnp
from jax.experimental import pallas as pl


def kernel(inputs):
    raise NotImplementedError("write your pallas kernel here")


# speedup vs baseline: 18.2782x; 18.2782x over previous
import functools

import jax
import jax.numpy as jnp
import numpy as np
from jax.experimental import pallas as pl

_K = 128


def _ce(x, bs, d, mh):
    n, m, db = x.shape
    no = n // (2 * d)
    y = x.reshape(no, 2, d, m, db)
    a, b = y[:, 0], y[:, 1]
    mx = jnp.maximum(a, b)
    mn = jnp.minimum(a, b)
    i0 = jax.lax.broadcasted_iota(jnp.int32, (no, 1, m, 1), 0)
    ic = jax.lax.broadcasted_iota(jnp.int32, (no, 1, m, 1), 2)
    blk_desc = (((i0 * (2 * d)) // bs) % 2) == 0
    dm = jnp.logical_xor(blk_desc, ic >= mh)
    first = jnp.where(dm, mx, mn)
    second = jnp.where(dm, mn, mx)
    return jnp.stack([first, second], axis=1).reshape(n, m, db)


def _topk_body(in_ref, out_ref):
    x = in_ref[0]
    s, db = x.shape
    m = s // _K
    x = x.reshape(_K, m, db)

    lg = _K.bit_length() - 1
    for k in range(lg):
        bs = 2 << k
        for j in range(k, -1, -1):
            x = _ce(x, bs, 1 << j, m // 2)

    while m > 1:
        mm = m // 2
        x = jnp.maximum(x[:, :mm], x[:, mm:])
        m = mm
        mh = m // 2 if m > 1 else 1
        for j in range(lg - 1, -1, -1):
            x = _ce(x, 2 * _K, 1 << j, mh)

    out_ref[...] = x.reshape(1, _K, db)


@jax.jit
def kernel(inputs):
    bsz, s, dim = inputs.shape
    db = 256
    grid = (bsz, dim // db)
    return pl.pallas_call(
        _topk_body,
        grid=grid,
        in_specs=[pl.BlockSpec((1, s, db), lambda b, j: (b, 0, j))],
        out_specs=pl.BlockSpec((1, _K, db), lambda b, j: (b, 0, j)),
        out_shape=jax.ShapeDtypeStruct((bsz, _K, dim), jnp.float32),
    )(inputs)


# speedup vs baseline: 18.2918x; 1.0007x over previous
import jax
import jax.numpy as jnp
from jax.experimental import pallas as pl

_K = 128


def _ce2(x, d):
    n = x.shape[0]
    rest = x.shape[1:]
    y = x.reshape(n // (2 * d), 2, d, *rest)
    a, b = y[:, 0], y[:, 1]
    return jnp.stack([jnp.maximum(a, b), jnp.minimum(a, b)], axis=1).reshape(
        n, *rest)


def _dir_desc(bs, m, mh):
    i0 = jax.lax.broadcasted_iota(jnp.int32, (_K, m, 1), 0)
    d = ((i0 // bs) % 2) == 0
    if mh < m:
        ic = jax.lax.broadcasted_iota(jnp.int32, (_K, m, 1), 1)
        d = jnp.logical_xor(d, ic >= mh)
    return d


def _topk_body(in_ref, out_ref):
    x = in_ref[0]
    s, db = x.shape
    m = s // _K
    x = x.reshape(_K, m, db)
    lg = _K.bit_length() - 1

    x = jnp.where(_dir_desc(2, m, m // 2), x, -x)

    for k in range(lg):
        for j in range(k, -1, -1):
            x = _ce2(x, 1 << j)
        if k + 1 < lg:
            flip = jnp.logical_xor(_dir_desc(2 << k, m, m),
                                   _dir_desc(4 << k, m, m))
            x = jnp.where(flip, -x, x)

    while m > 1:
        mm = m // 2
        x = jnp.maximum(x[:, :mm], -x[:, mm:])
        m = mm
        if m > 1:
            ic = jax.lax.broadcasted_iota(jnp.int32, (_K, m, 1), 1)
            x = jnp.where(ic >= m // 2, -x, x)
        for j in range(lg - 1, -1, -1):
            x = _ce2(x, 1 << j)

    out_ref[...] = x.reshape(1, _K, db)


@jax.jit
def kernel(inputs):
    bsz, s, dim = inputs.shape
    db = 256
    grid = (bsz, dim // db)
    return pl.pallas_call(
        _topk_body,
        grid=grid,
        in_specs=[pl.BlockSpec((1, s, db), lambda b, j: (b, 0, j))],
        out_specs=pl.BlockSpec((1, _K, db), lambda b, j: (b, 0, j)),
        out_shape=jax.ShapeDtypeStruct((bsz, _K, dim), jnp.float32),
    )(inputs)
